# Initial kernel scaffold; baseline (speedup 1.0000x reference)
import jax
import jax.numpy as jnp
from jax.experimental import pallas as pl


def kernel(x, edge_index, batch, W1, b1, W3, b3, W4, b4):
    raise NotImplementedError("write your pallas kernel here")


# speedup vs baseline: 2.3358x; 2.3358x over previous
import dataclasses
import functools

import jax
import jax.numpy as jnp
from jax import lax
from jax.experimental import pallas as pl
from jax.experimental.pallas import tpu as pltpu
from jax.experimental.pallas import tpu_sc as plsc

N = 10000
NPAD = 10240
ROWS = NPAD // 32
F = 256
FO = 128
E = 160000
EPAD = 163840
SCH = 256
NSCAN = EPAD // SCH
CH = 64
PMAX = 1024
G = 64

_mesh = plsc.VectorSubcoreMesh(core_axis_name="core", subcore_axis_name="subcore")
_f32 = jnp.float32
_i32 = jnp.int32

_cp = pltpu.CompilerParams()
if "needs_layout_passes" in pltpu.CompilerParams.__dataclass_fields__:
    _cp = dataclasses.replace(_cp, needs_layout_passes=False)


@functools.partial(
    pl.kernel,
    compiler_params=_cp,
    out_type=(
        jax.ShapeDtypeStruct((NPAD,), _f32),
        jax.ShapeDtypeStruct((256, 16), _f32),
    ),
    mesh=_mesh,
    scratch_types=[
        pltpu.VMEM((1, SCH), _i32),
        pltpu.VMEM((ROWS, 16), _f32),
        pltpu.VMEM((8, 16), _f32),
        pltpu.VMEM((ROWS,), _f32),
    ],
)
def _stats(dst_ref, batch_ref, dis_ref, cnt_ref, idxb, accd, accc, disb):
    c = lax.axis_index("core")
    s = lax.axis_index("subcore")
    wid = c * 16 + s
    base = wid * ROWS
    lanes = lax.iota(_i32, 16)
    ones = jnp.ones((16,), _f32)

    @pl.loop(0, ROWS)
    def _(r):
        accd[r, pl.ds(0, 16)] = jnp.zeros((16,), _f32)

    for r in range(8):
        accc[r, pl.ds(0, 16)] = jnp.zeros((16,), _f32)

    @pl.loop(0, NSCAN)
    def _(j):
        pltpu.sync_copy(dst_ref.at[pl.ds(j * SCH, SCH)], idxb.at[0])
        for i in range(SCH // 16):
            d = idxb[0, pl.ds(i * 16, 16)]
            loc = d - base
            m = (loc >= 0) & (loc < ROWS)
            loc = jnp.where(m, loc, 0)
            plsc.addupdate_scatter(accd, [loc, lanes], ones, mask=m)

    @pl.loop(0, NPAD // SCH)
    def _(j):
        pltpu.sync_copy(batch_ref.at[pl.ds(j * SCH, SCH)], idxb.at[0])
        for i in range(SCH // 16):
            g = idxb[0, pl.ds(i * 16, 16)]
            loc = g - wid * 8
            m = (loc >= 0) & (loc < 8)
            loc = jnp.where(m, loc, 0)
            plsc.addupdate_scatter(accc, [loc, lanes], ones, mask=m)

    @pl.loop(0, ROWS // 16)
    def _(k):
        rows = k * 16 + lanes
        deg = jnp.ones((16,), _f32)
        for col in range(16):
            deg = deg + plsc.load_gather(
                accd, [rows, lax.rem(lanes + col, jnp.full((16,), 16, _i32))])
        bits = lax.bitcast_convert_type(deg, _i32)
        y = lax.bitcast_convert_type(
            jnp.int32(0x5F3759DF) - lax.shift_right_arithmetic(bits, 1), _f32)
        for _ in range(4):
            y = y * (1.5 - 0.5 * deg * y * y)
        disb[pl.ds(k * 16, 16)] = y

    pltpu.sync_copy(disb, dis_ref.at[pl.ds(base, ROWS)])
    pltpu.sync_copy(accc, cnt_ref.at[pl.ds(wid * 8, 8)])


@functools.partial(
    pl.kernel,
    compiler_params=_cp,
    out_type=jax.ShapeDtypeStruct((NPAD, F), _f32),
    mesh=_mesh,
    scratch_types=[
        pltpu.VMEM((1, SCH), _i32),
        pltpu.VMEM((1, SCH), _i32),
        pltpu.VMEM((PMAX,), _i32),
        pltpu.VMEM((PMAX,), _i32),
        pltpu.VMEM((1, CH), _i32),
        pltpu.VMEM((1, CH), _i32),
        pltpu.VMEM((CH, F), _f32),
        pltpu.VMEM((CH, F), _f32),
        pltpu.VMEM((ROWS + 16, F), _f32),
        pltpu.SemaphoreType.DMA,
        pltpu.SemaphoreType.DMA,
    ],
)
def _agg(hs_ref, src_ref, dst_ref, out_ref, dbuf, sbuf, psrc, ploc,
         ist0, ist1, rows0, rows1, acc, sem0, sem1):
    c = lax.axis_index("core")
    s = lax.axis_index("subcore")
    wid = c * 16 + s
    base = wid * ROWS
    lanes = lax.iota(_i32, 16)

    @pl.loop(0, ROWS + 16)
    def _(r):
        for f in range(F // 16):
            acc[r, pl.ds(f * 16, 16)] = jnp.zeros((16,), _f32)

    def refill():
        @pl.loop(0, PMAX // 16)
        def _(q):
            psrc[pl.ds(q * 16, 16)] = q * 16 + lanes
            ploc[pl.ds(q * 16, 16)] = jnp.full((16,), ROWS, _i32)

    refill()

    def start(kk, ist, rows, sem):
        for q in range(CH // 16):
            ist[0, pl.ds(q * 16, 16)] = psrc[pl.ds(kk * CH + q * 16, 16)]
        pltpu.make_async_copy(hs_ref.at[ist.at[0]], rows, sem).start()

    def accum(kk, rows):
        @pl.loop(0, CH)
        def _(r):
            locv = plsc.load_gather(ploc, [jnp.full((16,), kk * CH + r, _i32)])
            for f in range(F // 16):
                plsc.addupdate_scatter(
                    acc, [locv, lanes + f * 16], rows[r, pl.ds(f * 16, 16)])

    def flush(off):
        nf = (off + CH - 1) // CH

        @pl.when(nf > 0)
        def _():
            start(0, ist0, rows0, sem0)

        @pl.when(nf > 1)
        def _():
            start(1, ist1, rows1, sem1)

        @pl.loop(0, (nf + 1) // 2)
        def _(t):
            pltpu.make_async_copy(hs_ref.at[ist0.at[0]], rows0, sem0).wait()
            accum(2 * t, rows0)

            @pl.when(2 * t + 2 < nf)
            def _():
                start(2 * t + 2, ist0, rows0, sem0)

            @pl.when(2 * t + 1 < nf)
            def _():
                pltpu.make_async_copy(hs_ref.at[ist1.at[0]], rows1, sem1).wait()
                accum(2 * t + 1, rows1)

                @pl.when(2 * t + 3 < nf)
                def _():
                    start(2 * t + 3, ist1, rows1, sem1)

        refill()

    def chunk_body(j, off):
        pltpu.sync_copy(dst_ref.at[pl.ds(j * SCH, SCH)], dbuf.at[0])
        pltpu.sync_copy(src_ref.at[pl.ds(j * SCH, SCH)], sbuf.at[0])

        def grp(i, off):
            d = dbuf[0, pl.ds(i * 16, 16)]
            v = sbuf[0, pl.ds(i * 16, 16)]
            loc = d - base
            m = (loc >= 0) & (loc < ROWS)
            plsc.store_compressed(psrc.at[pl.ds(off, 16)], v, mask=m)
            plsc.store_compressed(ploc.at[pl.ds(off, 16)], loc, mask=m)
            return off + plsc.all_reduce_population_count(m)[0]

        off = lax.fori_loop(0, SCH // 16, grp, off)

        @pl.when(off >= PMAX - SCH)
        def _():
            flush(off)

        return jnp.where(off >= PMAX - SCH, 0, off)

    off = lax.fori_loop(0, NSCAN, chunk_body, jnp.int32(0))
    flush(off)
    pltpu.sync_copy(acc.at[pl.ds(0, ROWS)], out_ref.at[pl.ds(base, ROWS)])


_RCH = 32


@functools.partial(
    pl.kernel,
    compiler_params=_cp,
    out_type=jax.ShapeDtypeStruct((32, 72, FO), _f32),
    mesh=_mesh,
    scratch_types=[
        pltpu.VMEM((_RCH, F), _f32),
        pltpu.VMEM((_RCH, F), _f32),
        pltpu.VMEM((_RCH, FO), _f32),
        pltpu.VMEM((_RCH,), _f32),
        pltpu.VMEM((_RCH,), _i32),
        pltpu.VMEM((FO,), _f32),
        pltpu.VMEM((FO,), _f32),
        pltpu.VMEM((72, FO), _f32),
    ],
)
def _zpool(agg_ref, hs_ref, dis_ref, b3_ref, b4_ref, eps_ref, batch_ref,
           part_ref, ab, hb, eb, db, bb, b3b, b4b, pacc):
    c = lax.axis_index("core")
    s = lax.axis_index("subcore")
    wid = c * 16 + s
    rbase = wid * (NPAD // 32)
    lanes = lax.iota(_i32, 16)

    @pl.loop(0, 72)
    def _(r):
        for f in range(FO // 16):
            pacc[r, pl.ds(f * 16, 16)] = jnp.zeros((16,), _f32)

    pltpu.sync_copy(b3_ref, b3b)
    pltpu.sync_copy(b4_ref, b4b)

    @pl.loop(0, NPAD // 32 // _RCH)
    def _(k):
        r0 = rbase + k * _RCH
        pltpu.sync_copy(agg_ref.at[pl.ds(r0, _RCH)], ab)
        pltpu.sync_copy(hs_ref.at[pl.ds(r0, _RCH)], hb)
        pltpu.sync_copy(eps_ref.at[pl.ds(r0, _RCH)], eb)
        pltpu.sync_copy(dis_ref.at[pl.ds(r0, _RCH)], db)
        pltpu.sync_copy(batch_ref.at[pl.ds(r0, _RCH)], bb)

        @pl.loop(0, _RCH)
        def _(r):
            dsv = plsc.load_gather(db, [jnp.full((16,), r, _i32)])
            gv = plsc.load_gather(bb, [jnp.full((16,), r, _i32)])
            for f in range(FO // 16):
                am = ab[r, pl.ds(f * 16, 16)]
                hm = hb[r, pl.ds(f * 16, 16)]
                al = ab[r, pl.ds(FO + f * 16, 16)]
                hl = hb[r, pl.ds(FO + f * 16, 16)]
                ev = eb[r, pl.ds(f * 16, 16)]
                mean = dsv * (am + hm) + b3b[pl.ds(f * 16, 16)]
                lstd = dsv * (al + hl) + b4b[pl.ds(f * 16, 16)]
                plsc.addupdate_scatter(
                    pacc, [gv, lanes + f * 16], mean + ev * jnp.exp(lstd))

    pltpu.sync_copy(pacc, part_ref.at[wid])


@functools.partial(
    pl.kernel,
    compiler_params=_cp,
    out_type=jax.ShapeDtypeStruct((G, FO), _f32),
    mesh=_mesh,
    scratch_types=[
        pltpu.VMEM((8, FO), _f32),
        pltpu.VMEM((8, 16), _f32),
        pltpu.VMEM((8, FO), _f32),
    ],
)
def _final(part_ref, cnt_ref, out_ref, pb, cb, ob):
    c = lax.axis_index("core")
    s = lax.axis_index("subcore")

    @pl.when((c == 0) & (s < 8))
    def _():
        for g in range(8):
            for f in range(FO // 16):
                ob[g, pl.ds(f * 16, 16)] = jnp.zeros((16,), _f32)

        @pl.loop(0, 32)
        def _(t):
            pltpu.sync_copy(part_ref.at[t].at[pl.ds(8 * s, 8)], pb)
            for g in range(8):
                for f in range(FO // 16):
                    ob[g, pl.ds(f * 16, 16)] += pb[g, pl.ds(f * 16, 16)]

        pltpu.sync_copy(cnt_ref.at[pl.ds(8 * s, 8)], cb)
        for g in range(8):
            tot = jnp.zeros((16,), _f32)
            for col in range(16):
                tot = tot + plsc.load_gather(
                    cb, [jnp.full((16,), g, _i32), jnp.full((16,), col, _i32)])
            rec = 1.0 / jnp.maximum(tot, 1.0)
            for f in range(FO // 16):
                ob[g, pl.ds(f * 16, 16)] = ob[g, pl.ds(f * 16, 16)] * rec

        pltpu.sync_copy(ob, out_ref.at[pl.ds(8 * s, 8)])


def _mm1_body(x_ref, w_ref, d_ref, o_ref):
    o_ref[...] = d_ref[...] * jnp.dot(
        x_ref[...], w_ref[...], preferred_element_type=_f32)


_mm1 = pl.pallas_call(
    _mm1_body,
    grid=(NPAD // 256,),
    in_specs=[
        pl.BlockSpec((256, F), lambda i: (i, 0)),
        pl.BlockSpec((F, F), lambda i: (0, 0)),
        pl.BlockSpec((256, 1), lambda i: (i, 0)),
    ],
    out_specs=pl.BlockSpec((256, F), lambda i: (i, 0)),
    out_shape=jax.ShapeDtypeStruct((NPAD, F), _f32),
)


def _mm2_body(a_ref, m_ref, d_ref, b_ref, w_ref, o_ref):
    d = d_ref[...]
    h = jnp.maximum(d * (a_ref[...] + m_ref[...]) + b_ref[...], 0.0)
    o_ref[...] = d * jnp.dot(h, w_ref[...], preferred_element_type=_f32)


_mm2 = pl.pallas_call(
    _mm2_body,
    grid=(NPAD // 256,),
    in_specs=[
        pl.BlockSpec((256, F), lambda i: (i, 0)),
        pl.BlockSpec((256, F), lambda i: (i, 0)),
        pl.BlockSpec((256, 1), lambda i: (i, 0)),
        pl.BlockSpec((1, F), lambda i: (0, 0)),
        pl.BlockSpec((F, F), lambda i: (0, 0)),
    ],
    out_specs=pl.BlockSpec((256, F), lambda i: (i, 0)),
    out_shape=jax.ShapeDtypeStruct((NPAD, F), _f32),
)


def kernel(x, edge_index, batch, W1, b1, W3, b3, W4, b4):
    src = edge_index[0].astype(_i32)
    dst = edge_index[1].astype(_i32)
    srcp = jnp.concatenate([src, jnp.zeros((EPAD - E,), _i32)])
    dstp = jnp.concatenate([dst, jnp.full((EPAD - E,), NPAD, _i32)])
    batchp = jnp.concatenate(
        [batch.astype(_i32), jnp.full((NPAD - N,), G, _i32)])

    dis, cnt = _stats(dstp, batchp)
    dis2d = dis.reshape(NPAD, 1)

    hs0 = _mm1(x, W1, dis2d)
    agg1 = _agg(hs0, srcp, dstp)

    Wc = jnp.concatenate([W3, W4], axis=1)
    hs1 = _mm2(agg1, hs0, dis2d, b1.reshape(1, F), Wc)
    agg2 = _agg(hs1, srcp, dstp)

    eps = jax.random.normal(jax.random.key(42), (N, FO), _f32)
    epsp = jnp.pad(eps, ((0, NPAD - N), (0, 0)))
    part = _zpool(agg2, hs1, dis, b3, b4, epsp, batchp)
    return _final(part, cnt)


# speedup vs baseline: 4.1449x; 1.7745x over previous
import dataclasses
import functools

import jax
import jax.numpy as jnp
from jax import lax
from jax.experimental import pallas as pl
from jax.experimental.pallas import tpu as pltpu
from jax.experimental.pallas import tpu_sc as plsc

N = 10000
NPAD = 10240
ROWS = NPAD // 32
F = 256
FO = 128
E = 160000
EPAD = 163840
SCH = 256
NSCAN = EPAD // SCH
CH = 64
PMAX = 1024
G = 64

_mesh = plsc.VectorSubcoreMesh(core_axis_name="core", subcore_axis_name="subcore")
_f32 = jnp.float32
_i32 = jnp.int32

_cp = pltpu.CompilerParams()
if "needs_layout_passes" in pltpu.CompilerParams.__dataclass_fields__:
    _cp = dataclasses.replace(_cp, needs_layout_passes=False)


@functools.partial(
    pl.kernel,
    compiler_params=_cp,
    out_type=(
        jax.ShapeDtypeStruct((NPAD,), _f32),
        jax.ShapeDtypeStruct((256, 16), _f32),
    ),
    mesh=_mesh,
    scratch_types=[
        pltpu.VMEM((1, SCH), _i32),
        pltpu.VMEM((1, SCH), _i32),
        pltpu.VMEM((ROWS, 16), _f32),
        pltpu.VMEM((8, 16), _f32),
        pltpu.VMEM((ROWS,), _f32),
        pltpu.SemaphoreType.DMA,
        pltpu.SemaphoreType.DMA,
    ],
)
def _stats(dst_ref, batch_ref, dis_ref, cnt_ref, idxa, idxb2, accd, accc,
           disb, sema, semb):
    c = lax.axis_index("core")
    s = lax.axis_index("subcore")
    wid = c * 16 + s
    base = wid * ROWS
    lanes = lax.iota(_i32, 16)
    ones = jnp.ones((16,), _f32)

    @pl.loop(0, ROWS)
    def _(r):
        accd[r, pl.ds(0, 16)] = jnp.zeros((16,), _f32)

    for r in range(8):
        accc[r, pl.ds(0, 16)] = jnp.zeros((16,), _f32)

    def hist(ib, acc, nbins, bbase):
        for i in range(SCH // 16):
            d = ib[0, pl.ds(i * 16, 16)]
            loc = d - bbase
            m = (loc >= 0) & (loc < nbins)
            loc = jnp.where(m, loc, 0)
            plsc.addupdate_scatter(acc, [loc, lanes], ones, mask=m)

    def start(ref, j, ib, sem):
        pltpu.make_async_copy(ref.at[pl.ds(j * SCH, SCH)], ib.at[0], sem).start()

    def wait(ref, j, ib, sem):
        pltpu.make_async_copy(ref.at[pl.ds(j * SCH, SCH)], ib.at[0], sem).wait()

    start(dst_ref, 0, idxa, sema)
    start(dst_ref, 1, idxb2, semb)

    @pl.loop(0, NSCAN // 2)
    def _(t):
        wait(dst_ref, 2 * t, idxa, sema)
        hist(idxa, accd, ROWS, base)

        @pl.when(2 * t + 2 < NSCAN)
        def _():
            start(dst_ref, 2 * t + 2, idxa, sema)

        wait(dst_ref, 2 * t + 1, idxb2, semb)
        hist(idxb2, accd, ROWS, base)

        @pl.when(2 * t + 3 < NSCAN)
        def _():
            start(dst_ref, 2 * t + 3, idxb2, semb)

    start(batch_ref, 0, idxa, sema)
    start(batch_ref, 1, idxb2, semb)

    @pl.loop(0, NPAD // SCH // 2)
    def _(t):
        wait(batch_ref, 2 * t, idxa, sema)
        hist(idxa, accc, 8, wid * 8)

        @pl.when(2 * t + 2 < NPAD // SCH)
        def _():
            start(batch_ref, 2 * t + 2, idxa, sema)

        wait(batch_ref, 2 * t + 1, idxb2, semb)
        hist(idxb2, accc, 8, wid * 8)

        @pl.when(2 * t + 3 < NPAD // SCH)
        def _():
            start(batch_ref, 2 * t + 3, idxb2, semb)

    @pl.loop(0, ROWS // 16)
    def _(k):
        rows = k * 16 + lanes
        deg = jnp.ones((16,), _f32)
        for col in range(16):
            deg = deg + plsc.load_gather(
                accd, [rows, lax.rem(lanes + col, jnp.full((16,), 16, _i32))])
        bits = lax.bitcast_convert_type(deg, _i32)
        y = lax.bitcast_convert_type(
            jnp.int32(0x5F3759DF) - lax.shift_right_arithmetic(bits, 1), _f32)
        for _ in range(4):
            y = y * (1.5 - 0.5 * deg * y * y)
        disb[pl.ds(k * 16, 16)] = y

    pltpu.sync_copy(disb, dis_ref.at[pl.ds(base, ROWS)])
    pltpu.sync_copy(accc, cnt_ref.at[pl.ds(wid * 8, 8)])


@functools.partial(
    pl.kernel,
    compiler_params=_cp,
    out_type=jax.ShapeDtypeStruct((NPAD, F), _f32),
    mesh=_mesh,
    scratch_types=[
        pltpu.VMEM((1, SCH), _i32),
        pltpu.VMEM((1, SCH), _i32),
        pltpu.VMEM((1, SCH), _i32),
        pltpu.VMEM((1, SCH), _i32),
        pltpu.VMEM((PMAX,), _i32),
        pltpu.VMEM((PMAX,), _i32),
        pltpu.VMEM((1, CH), _i32),
        pltpu.VMEM((1, CH), _i32),
        pltpu.VMEM((CH, F), _f32),
        pltpu.VMEM((CH, F), _f32),
        pltpu.VMEM((ROWS + 16, F), _f32),
        pltpu.SemaphoreType.DMA,
        pltpu.SemaphoreType.DMA,
        pltpu.SemaphoreType.DMA,
        pltpu.SemaphoreType.DMA,
    ],
)
def _agg(hs_ref, src_ref, dst_ref, out_ref, dbufa, sbufa, dbufb, sbufb,
         psrc, ploc, ist0, ist1, rows0, rows1, acc, sem0, sem1, sema, semb):
    c = lax.axis_index("core")
    s = lax.axis_index("subcore")
    wid = c * 16 + s
    base = wid * ROWS
    lanes = lax.iota(_i32, 16)

    @pl.loop(0, ROWS + 16)
    def _(r):
        for f in range(F // 16):
            acc[r, pl.ds(f * 16, 16)] = jnp.zeros((16,), _f32)

    def refill():
        @pl.loop(0, PMAX // 16)
        def _(q):
            psrc[pl.ds(q * 16, 16)] = q * 16 + lanes
            ploc[pl.ds(q * 16, 16)] = jnp.full((16,), ROWS, _i32)

    refill()

    def start(kk, ist, rows, sem):
        for q in range(CH // 16):
            ist[0, pl.ds(q * 16, 16)] = psrc[pl.ds(kk * CH + q * 16, 16)]
        pltpu.make_async_copy(hs_ref.at[ist.at[0]], rows, sem).start()

    def accum(kk, rows):
        @pl.loop(0, CH)
        def _(r):
            locv = plsc.load_gather(ploc, [jnp.full((16,), kk * CH + r, _i32)])
            for f in range(F // 16):
                plsc.addupdate_scatter(
                    acc, [locv, lanes + f * 16], rows[r, pl.ds(f * 16, 16)])

    def flush(off):
        nf = (off + CH - 1) // CH

        @pl.when(nf > 0)
        def _():
            start(0, ist0, rows0, sem0)

        @pl.when(nf > 1)
        def _():
            start(1, ist1, rows1, sem1)

        @pl.loop(0, (nf + 1) // 2)
        def _(t):
            pltpu.make_async_copy(hs_ref.at[ist0.at[0]], rows0, sem0).wait()
            accum(2 * t, rows0)

            @pl.when(2 * t + 2 < nf)
            def _():
                start(2 * t + 2, ist0, rows0, sem0)

            @pl.when(2 * t + 1 < nf)
            def _():
                pltpu.make_async_copy(hs_ref.at[ist1.at[0]], rows1, sem1).wait()
                accum(2 * t + 1, rows1)

                @pl.when(2 * t + 3 < nf)
                def _():
                    start(2 * t + 3, ist1, rows1, sem1)

        refill()

    def startscan(j, db, sb, sem):
        pltpu.make_async_copy(dst_ref.at[pl.ds(j * SCH, SCH)], db.at[0], sem).start()
        pltpu.make_async_copy(src_ref.at[pl.ds(j * SCH, SCH)], sb.at[0], sem).start()

    def waitscan(j, db, sb, sem):
        pltpu.make_async_copy(dst_ref.at[pl.ds(j * SCH, SCH)], db.at[0], sem).wait()
        pltpu.make_async_copy(src_ref.at[pl.ds(j * SCH, SCH)], sb.at[0], sem).wait()

    def process(j, db, sb, off):
        def grp(i, off):
            d = db[0, pl.ds(i * 16, 16)]
            v = sb[0, pl.ds(i * 16, 16)]
            loc = d - base
            m = (loc >= 0) & (loc < ROWS)
            plsc.store_compressed(psrc.at[pl.ds(off, 16)], v, mask=m)
            plsc.store_compressed(ploc.at[pl.ds(off, 16)], loc, mask=m)
            return off + plsc.all_reduce_population_count(m)[0]

        off = lax.fori_loop(0, SCH // 16, grp, off)

        @pl.when(off >= PMAX - SCH)
        def _():
            flush(off)

        return jnp.where(off >= PMAX - SCH, 0, off)

    startscan(0, dbufa, sbufa, sema)
    startscan(1, dbufb, sbufb, semb)

    def chunk_body(t, off):
        waitscan(2 * t, dbufa, sbufa, sema)
        off = process(2 * t, dbufa, sbufa, off)

        @pl.when(2 * t + 2 < NSCAN)
        def _():
            startscan(2 * t + 2, dbufa, sbufa, sema)

        waitscan(2 * t + 1, dbufb, sbufb, semb)
        off = process(2 * t + 1, dbufb, sbufb, off)

        @pl.when(2 * t + 3 < NSCAN)
        def _():
            startscan(2 * t + 3, dbufb, sbufb, semb)

        return off

    off = lax.fori_loop(0, NSCAN // 2, chunk_body, jnp.int32(0))
    flush(off)
    pltpu.sync_copy(acc.at[pl.ds(0, ROWS)], out_ref.at[pl.ds(base, ROWS)])


_RCH = 32


@functools.partial(
    pl.kernel,
    compiler_params=_cp,
    out_type=jax.ShapeDtypeStruct((32, 72, FO), _f32),
    mesh=_mesh,
    scratch_types=[
        pltpu.VMEM((_RCH, F), _f32),
        pltpu.VMEM((_RCH, F), _f32),
        pltpu.VMEM((_RCH, FO), _f32),
        pltpu.VMEM((_RCH,), _f32),
        pltpu.VMEM((_RCH,), _i32),
        pltpu.VMEM((FO,), _f32),
        pltpu.VMEM((FO,), _f32),
        pltpu.VMEM((72, FO), _f32),
    ],
)
def _zpool(agg_ref, hs_ref, dis_ref, b3_ref, b4_ref, eps_ref, batch_ref,
           part_ref, ab, hb, eb, db, bb, b3b, b4b, pacc):
    c = lax.axis_index("core")
    s = lax.axis_index("subcore")
    wid = c * 16 + s
    rbase = wid * (NPAD // 32)
    lanes = lax.iota(_i32, 16)

    @pl.loop(0, 72)
    def _(r):
        for f in range(FO // 16):
            pacc[r, pl.ds(f * 16, 16)] = jnp.zeros((16,), _f32)

    pltpu.sync_copy(b3_ref, b3b)
    pltpu.sync_copy(b4_ref, b4b)

    @pl.loop(0, NPAD // 32 // _RCH)
    def _(k):
        r0 = rbase + k * _RCH
        pltpu.sync_copy(agg_ref.at[pl.ds(r0, _RCH)], ab)
        pltpu.sync_copy(hs_ref.at[pl.ds(r0, _RCH)], hb)
        pltpu.sync_copy(eps_ref.at[pl.ds(r0, _RCH)], eb)
        pltpu.sync_copy(dis_ref.at[pl.ds(r0, _RCH)], db)
        pltpu.sync_copy(batch_ref.at[pl.ds(r0, _RCH)], bb)

        @pl.loop(0, _RCH)
        def _(r):
            dsv = plsc.load_gather(db, [jnp.full((16,), r, _i32)])
            gv = plsc.load_gather(bb, [jnp.full((16,), r, _i32)])
            for f in range(FO // 16):
                am = ab[r, pl.ds(f * 16, 16)]
                hm = hb[r, pl.ds(f * 16, 16)]
                al = ab[r, pl.ds(FO + f * 16, 16)]
                hl = hb[r, pl.ds(FO + f * 16, 16)]
                ev = eb[r, pl.ds(f * 16, 16)]
                mean = dsv * (am + hm) + b3b[pl.ds(f * 16, 16)]
                lstd = dsv * (al + hl) + b4b[pl.ds(f * 16, 16)]
                plsc.addupdate_scatter(
                    pacc, [gv, lanes + f * 16], mean + ev * jnp.exp(lstd))

    pltpu.sync_copy(pacc, part_ref.at[wid])


@functools.partial(
    pl.kernel,
    compiler_params=_cp,
    out_type=jax.ShapeDtypeStruct((G, FO), _f32),
    mesh=_mesh,
    scratch_types=[
        pltpu.VMEM((8, FO), _f32),
        pltpu.VMEM((8, 16), _f32),
        pltpu.VMEM((8, FO), _f32),
    ],
)
def _final(part_ref, cnt_ref, out_ref, pb, cb, ob):
    c = lax.axis_index("core")
    s = lax.axis_index("subcore")

    @pl.when((c == 0) & (s < 8))
    def _():
        for g in range(8):
            for f in range(FO // 16):
                ob[g, pl.ds(f * 16, 16)] = jnp.zeros((16,), _f32)

        @pl.loop(0, 32)
        def _(t):
            pltpu.sync_copy(part_ref.at[t].at[pl.ds(8 * s, 8)], pb)
            for g in range(8):
                for f in range(FO // 16):
                    ob[g, pl.ds(f * 16, 16)] += pb[g, pl.ds(f * 16, 16)]

        pltpu.sync_copy(cnt_ref.at[pl.ds(8 * s, 8)], cb)
        for g in range(8):
            tot = jnp.zeros((16,), _f32)
            for col in range(16):
                tot = tot + plsc.load_gather(
                    cb, [jnp.full((16,), g, _i32), jnp.full((16,), col, _i32)])
            rec = 1.0 / jnp.maximum(tot, 1.0)
            for f in range(FO // 16):
                ob[g, pl.ds(f * 16, 16)] = ob[g, pl.ds(f * 16, 16)] * rec

        pltpu.sync_copy(ob, out_ref.at[pl.ds(8 * s, 8)])


def _mm1_body(x_ref, w_ref, d_ref, o_ref):
    o_ref[...] = d_ref[...] * jnp.dot(
        x_ref[...], w_ref[...], preferred_element_type=_f32)


_mm1 = pl.pallas_call(
    _mm1_body,
    grid=(NPAD // 256,),
    in_specs=[
        pl.BlockSpec((256, F), lambda i: (i, 0)),
        pl.BlockSpec((F, F), lambda i: (0, 0)),
        pl.BlockSpec((256, 1), lambda i: (i, 0)),
    ],
    out_specs=pl.BlockSpec((256, F), lambda i: (i, 0)),
    out_shape=jax.ShapeDtypeStruct((NPAD, F), _f32),
)


def _mm2_body(a_ref, m_ref, d_ref, b_ref, w_ref, o_ref):
    d = d_ref[...]
    h = jnp.maximum(d * (a_ref[...] + m_ref[...]) + b_ref[...], 0.0)
    o_ref[...] = d * jnp.dot(h, w_ref[...], preferred_element_type=_f32)


_mm2 = pl.pallas_call(
    _mm2_body,
    grid=(NPAD // 256,),
    in_specs=[
        pl.BlockSpec((256, F), lambda i: (i, 0)),
        pl.BlockSpec((256, F), lambda i: (i, 0)),
        pl.BlockSpec((256, 1), lambda i: (i, 0)),
        pl.BlockSpec((1, F), lambda i: (0, 0)),
        pl.BlockSpec((F, F), lambda i: (0, 0)),
    ],
    out_specs=pl.BlockSpec((256, F), lambda i: (i, 0)),
    out_shape=jax.ShapeDtypeStruct((NPAD, F), _f32),
)


def kernel(x, edge_index, batch, W1, b1, W3, b3, W4, b4):
    src = edge_index[0].astype(_i32)
    dst = edge_index[1].astype(_i32)
    srcp = jnp.concatenate([src, jnp.zeros((EPAD - E,), _i32)])
    dstp = jnp.concatenate([dst, jnp.full((EPAD - E,), NPAD, _i32)])
    batchp = jnp.concatenate(
        [batch.astype(_i32), jnp.full((NPAD - N,), G, _i32)])

    dis, cnt = _stats(dstp, batchp)
    dis2d = dis.reshape(NPAD, 1)

    hs0 = _mm1(x, W1, dis2d)
    agg1 = _agg(hs0, srcp, dstp)

    Wc = jnp.concatenate([W3, W4], axis=1)
    hs1 = _mm2(agg1, hs0, dis2d, b1.reshape(1, F), Wc)
    agg2 = _agg(hs1, srcp, dstp)

    eps = jax.random.normal(jax.random.key(42), (N, FO), _f32)
    epsp = jnp.pad(eps, ((0, NPAD - N), (0, 0)))
    part = _zpool(agg2, hs1, dis, b3, b4, epsp, batchp)
    return _final(part, cnt)
